# baseline (speedup 1.0000x reference)
import functools

import jax
import jax.numpy as jnp
from jax import lax
from jax.experimental import pallas as pl
from jax.experimental.pallas import tpu as pltpu
from jax.experimental.pallas import tpu_sc as plsc

_NUM_NODES = 1000000
_D = 16
_B = 16384
_NS = 16
_PER_TILE = _B // _NS
_CH = 128
_NCH = _PER_TILE // _CH
_IDX_ROWS_PER_TILE = _PER_TILE // _CH

_mesh = plsc.VectorSubcoreMesh(core_axis_name="c", subcore_axis_name="s")


@functools.partial(
    pl.kernel,
    out_type=jax.ShapeDtypeStruct((_B, _D), jnp.float32),
    mesh=_mesh,
    scratch_types=[
        pltpu.VMEM_SHARED((_NUM_NODES,), jnp.int32),
        pltpu.VMEM_SHARED((_B, _D), jnp.float32),
        pltpu.VMEM_SHARED((_B,), jnp.float32),
        pltpu.VMEM((_NCH, _CH), jnp.int32),
        pltpu.VMEM((_NCH, _CH), jnp.int32),
        pltpu.VMEM((_NCH, _CH), jnp.int32),
        pltpu.VMEM((_NCH, _CH, _D), jnp.float32),
        pltpu.VMEM((_NCH, _CH, _D), jnp.float32),
        pltpu.VMEM((_NCH, _CH), jnp.float32),
        pltpu.VMEM((_CH, _D), jnp.float32),
        pltpu.VMEM((_CH,), jnp.float32),
        pltpu.VMEM((_CH,), jnp.float32),
    ],
    compiler_params=pltpu.CompilerParams(use_tc_tiling_on_sc=False),
)
def _sc_embed_update(weight, node2d, ids2d, update, zrows, zcol, ocol, outw,
                     emb, slot_tab, acc, cnt, idx_v, ids_v, rep_v, upd_v,
                     val_v, cnt_v, zb_v, zc_v, oc_v):
    c = lax.axis_index("c")
    s = lax.axis_index("s")
    base = s * _PER_TILE
    rowbase = s * _IDX_ROWS_PER_TILE

    @pl.when(c == 0)
    def _():
        pltpu.sync_copy(node2d.at[pl.ds(rowbase, _NCH)], idx_v)
        pltpu.sync_copy(ids2d.at[pl.ds(rowbase, _NCH)], ids_v)
        pltpu.sync_copy(zrows, zb_v)
        pltpu.sync_copy(zcol, zc_v)
        pltpu.sync_copy(ocol, oc_v)
        for j in range(_NCH):
            pltpu.sync_copy(update.at[pl.ds(base + j * _CH, _CH)], upd_v.at[j])
            pltpu.sync_copy(zb_v, acc.at[pl.ds(base + j * _CH, _CH)])
            pltpu.sync_copy(zc_v, cnt.at[pl.ds(base + j * _CH, _CH)])
            pltpu.sync_copy(ids_v.at[j], slot_tab.at[idx_v.at[j]])

    @pl.when(c == 1)
    def _():
        pltpu.sync_copy(node2d.at[pl.ds(rowbase, _NCH)], idx_v)
        for j in range(_NCH):
            pltpu.sync_copy(weight.at[idx_v.at[j]], val_v.at[j])
            pltpu.sync_copy(val_v.at[j], emb.at[pl.ds(base + j * _CH, _CH)])

    plsc.subcore_barrier()

    @pl.when(c == 0)
    def _():
        for j in range(_NCH):
            pltpu.sync_copy(slot_tab.at[idx_v.at[j]], rep_v.at[j])
        for j in range(_NCH):
            pltpu.sync_copy(upd_v.at[j], acc.at[rep_v.at[j]], add=True)
            pltpu.sync_copy(oc_v, cnt.at[rep_v.at[j]], add=True)

    plsc.subcore_barrier()

    @pl.when(c == 0)
    def _():
        for j in range(_NCH):
            pltpu.sync_copy(acc.at[rep_v.at[j]], val_v.at[j])
            pltpu.sync_copy(cnt.at[rep_v.at[j]], cnt_v.at[j])
        for j in range(_NCH):
            def _div(g, _):
                gbase = g * _D
                recip = 1.0 / cnt_v[j, pl.ds(gbase, _D)]
                for k in range(_D):
                    val_v[j, gbase + k, :] = val_v[j, gbase + k, :] * recip[k]
                return 0
            lax.fori_loop(0, _CH // _D, _div, 0)
        for j in range(_NCH):
            pltpu.sync_copy(val_v.at[j], outw.at[idx_v.at[j]])


def kernel(weight, node, update):
    node2d = node.reshape(_B // _CH, _CH)
    ids2d = jnp.arange(_B, dtype=jnp.int32).reshape(_B // _CH, _CH)
    zrows = jnp.zeros((_CH, _D), jnp.float32)
    zcol = jnp.zeros((_CH,), jnp.float32)
    ocol = jnp.ones((_CH,), jnp.float32)
    outw = jax.new_ref(weight)
    emb = _sc_embed_update(weight, node2d, ids2d, update, zrows, zcol, ocol,
                           outw)
    return emb, jax.freeze(outw)
